# Initial kernel scaffold; baseline (speedup 1.0000x reference)
import jax
import jax.numpy as jnp
import numpy as np
from jax import lax
from jax.experimental import pallas as pl
from jax.experimental.pallas import tpu as pltpu

_SCORE_T = 0.05
_NMS_T = 0.6
_MAXDET = 300

_NB = 10
_NPAD = 20480
_ROWS = _NPAD // 128


def _score_body(cls_ref, cent_ref, work_ref, lab_ref):
    cls = cls_ref[:, 0]
    m = jnp.max(cls, axis=-1)
    iota_c = lax.broadcasted_iota(jnp.float32, cls.shape, 2)
    lab = jnp.min(jnp.where(cls == m[:, :, None], iota_c, 1e9), axis=-1)
    c0 = cent_ref[:, 0, 0]
    w = jnp.where(m > _SCORE_T, jnp.sqrt(c0 * m), -jnp.inf)
    work_ref[:, 0, 0] = w
    lab_ref[:, 0, 0] = lab


def _nms_body(w_ref, b_ref, l_ref, out_ref):
    B = w_ref.shape[0]
    shp = (B, _ROWS, 128)
    w0 = w_ref[...].reshape(shp)
    x1 = b_ref[:, 0, :].reshape(shp)
    y1 = b_ref[:, 1, :].reshape(shp)
    x2 = b_ref[:, 2, :].reshape(shp)
    y2 = b_ref[:, 3, :].reshape(shp)
    lab = l_ref[...].reshape(shp)
    areas = jnp.maximum(x2 - x1, 0.0) * jnp.maximum(y2 - y1, 0.0)
    idxf = (lax.broadcasted_iota(jnp.float32, shp, 1) * 128.0
            + lax.broadcasted_iota(jnp.float32, shp, 2))

    def red(op, v):
        return op(op(v, axis=2, keepdims=True), axis=1, keepdims=True)

    def ext(sel, v):
        return red(jnp.sum, jnp.where(sel, v, 0.0))

    def body(i, w):
        m = red(jnp.max, w)
        eq = w == m
        bidx = red(jnp.min, jnp.where(eq, idxf, 3e7))
        sel = idxf == bidx
        bx1 = ext(sel, x1)
        by1 = ext(sel, y1)
        bx2 = ext(sel, x2)
        by2 = ext(sel, y2)
        bar = ext(sel, areas)
        blab = ext(sel, lab)
        valid = m > -jnp.inf
        mask = lambda v: jnp.where(valid, v, -1.0)
        row = jnp.concatenate(
            [mask(bx1), mask(by1), mask(bx2), mask(by2), mask(m), mask(blab),
             valid.astype(jnp.float32), jnp.zeros_like(m)], axis=-1)
        out_ref[:, pl.ds(i, 1), :] = row
        ix1 = jnp.maximum(x1, bx1)
        iy1 = jnp.maximum(y1, by1)
        ix2 = jnp.minimum(x2, bx2)
        iy2 = jnp.minimum(y2, by2)
        inter = jnp.maximum(ix2 - ix1, 0.0) * jnp.maximum(iy2 - iy1, 0.0)
        union = areas + bar - inter
        iou = jnp.where(union > 0.0, inter / union, 0.0)
        supp = (iou > _NMS_T) | sel
        return jnp.where(supp & valid, -jnp.inf, w)

    lax.fori_loop(0, _MAXDET, body, w0)


@jax.jit
def kernel(boxes, classification, centerness):
    B, N, C = classification.shape
    Bn = N // _NB
    cls_r = classification.reshape(B, _NB, Bn, C)
    cent_r = centerness.reshape(B, _NB, 1, Bn)

    work, labf = pl.pallas_call(
        _score_body,
        grid=(_NB,),
        in_specs=[
            pl.BlockSpec((B, 1, Bn, C), lambda i: (0, i, 0, 0)),
            pl.BlockSpec((B, 1, 1, Bn), lambda i: (0, i, 0, 0)),
        ],
        out_specs=[
            pl.BlockSpec((B, 1, 1, Bn), lambda i: (0, i, 0, 0)),
            pl.BlockSpec((B, 1, 1, Bn), lambda i: (0, i, 0, 0)),
        ],
        out_shape=[
            jax.ShapeDtypeStruct((B, _NB, 1, Bn), jnp.float32),
            jax.ShapeDtypeStruct((B, _NB, 1, Bn), jnp.float32),
        ],
    )(cls_r, cent_r)

    work = work.reshape(B, N)
    labf = labf.reshape(B, N)
    pad = _NPAD - N
    workp = jnp.pad(work, ((0, 0), (0, pad)), constant_values=-np.inf)
    labp = jnp.pad(labf, ((0, 0), (0, pad)))
    boxes_t = jnp.pad(boxes.transpose(0, 2, 1), ((0, 0), (0, 0), (0, pad)))

    misc = pl.pallas_call(
        _nms_body,
        in_specs=[
            pl.BlockSpec(memory_space=pltpu.VMEM),
            pl.BlockSpec(memory_space=pltpu.VMEM),
            pl.BlockSpec(memory_space=pltpu.VMEM),
        ],
        out_specs=pl.BlockSpec(memory_space=pltpu.VMEM),
        out_shape=jax.ShapeDtypeStruct((B, 304, 8), jnp.float32),
    )(workp, boxes_t, labp)

    sel = misc[:, :_MAXDET, :]
    out_boxes = sel[:, :, 0:4]
    out_scores = sel[:, :, 4]
    out_labels = sel[:, :, 5].astype(jnp.int32)
    return out_boxes, out_scores, out_labels


# speedup vs baseline: 1.9761x; 1.9761x over previous
import jax
import jax.numpy as jnp
import numpy as np
from jax import lax
from jax.experimental import pallas as pl
from jax.experimental.pallas import tpu as pltpu

_SCORE_T = 0.05
_NMS_T = 0.6
_MAXDET = 300

_NB = 10
_NPAD = 20480
_ROWS = _NPAD // 128


def _score_body(cls_ref, cent_ref, work_ref, lab_ref):
    cls = cls_ref[:, 0]
    m = jnp.max(cls, axis=-1)
    iota_c = lax.broadcasted_iota(jnp.int32, cls.shape, 2).astype(jnp.float32)
    lab = jnp.min(jnp.where(cls == m[:, :, None], iota_c, 1e9), axis=-1)
    c0 = cent_ref[:, 0, 0]
    w = jnp.where(m > _SCORE_T, jnp.sqrt(c0 * m), -jnp.inf)
    work_ref[:, 0, 0] = w
    lab_ref[:, 0, 0] = lab


def _nms_body(w_ref, b_ref, l_ref, out_ref):
    B = w_ref.shape[0]
    shp = (B, _ROWS, 128)
    w0 = w_ref[...].reshape(shp)
    x1 = b_ref[:, 0, :].reshape(shp)
    y1 = b_ref[:, 1, :].reshape(shp)
    x2 = b_ref[:, 2, :].reshape(shp)
    y2 = b_ref[:, 3, :].reshape(shp)
    lab = l_ref[...].reshape(shp)
    areas = jnp.maximum(x2 - x1, 0.0) * jnp.maximum(y2 - y1, 0.0)
    idxf = (lax.broadcasted_iota(jnp.int32, shp, 1) * 128
            + lax.broadcasted_iota(jnp.int32, shp, 2)).astype(jnp.float32)

    def red(op, v):
        return op(op(v, axis=2, keepdims=True), axis=1, keepdims=True)

    def ext(sel, v):
        return red(jnp.sum, jnp.where(sel, v, 0.0))

    def body(i, w):
        m = red(jnp.max, w)
        eq = w == m
        bidx = red(jnp.min, jnp.where(eq, idxf, 3e7))
        sel = idxf == bidx
        bx1 = ext(sel, x1)
        by1 = ext(sel, y1)
        bx2 = ext(sel, x2)
        by2 = ext(sel, y2)
        bar = ext(sel, areas)
        blab = ext(sel, lab)
        valid = m > -jnp.inf
        mask = lambda v: jnp.where(valid, v, -1.0)
        row = jnp.concatenate(
            [mask(bx1), mask(by1), mask(bx2), mask(by2), mask(m), mask(blab),
             valid.astype(jnp.float32), jnp.zeros_like(m)], axis=-1)
        out_ref[:, pl.ds(i, 1), :] = row
        ix1 = jnp.maximum(x1, bx1)
        iy1 = jnp.maximum(y1, by1)
        ix2 = jnp.minimum(x2, bx2)
        iy2 = jnp.minimum(y2, by2)
        inter = jnp.maximum(ix2 - ix1, 0.0) * jnp.maximum(iy2 - iy1, 0.0)
        union = areas + bar - inter
        iou = jnp.where(union > 0.0, inter / union, 0.0)
        supp = (iou > _NMS_T) | sel
        return jnp.where(supp & valid, -jnp.inf, w)

    lax.fori_loop(0, _MAXDET, body, w0)


@jax.jit
def kernel(boxes, classification, centerness):
    B, N, C = classification.shape
    Bn = N // _NB
    cls_r = classification.reshape(B, _NB, Bn, C)
    cent_r = centerness.reshape(B, _NB, 1, Bn)

    work, labf = pl.pallas_call(
        _score_body,
        grid=(_NB,),
        in_specs=[
            pl.BlockSpec((B, 1, Bn, C), lambda i: (0, i, 0, 0)),
            pl.BlockSpec((B, 1, 1, Bn), lambda i: (0, i, 0, 0)),
        ],
        out_specs=[
            pl.BlockSpec((B, 1, 1, Bn), lambda i: (0, i, 0, 0)),
            pl.BlockSpec((B, 1, 1, Bn), lambda i: (0, i, 0, 0)),
        ],
        out_shape=[
            jax.ShapeDtypeStruct((B, _NB, 1, Bn), jnp.float32),
            jax.ShapeDtypeStruct((B, _NB, 1, Bn), jnp.float32),
        ],
    )(cls_r, cent_r)

    work = work.reshape(B, N)
    labf = labf.reshape(B, N)
    pad = _NPAD - N
    workp = jnp.pad(work, ((0, 0), (0, pad)), constant_values=-np.inf)
    labp = jnp.pad(labf, ((0, 0), (0, pad)))
    boxes_t = jnp.pad(boxes.transpose(0, 2, 1), ((0, 0), (0, 0), (0, pad)))

    misc = pl.pallas_call(
        _nms_body,
        in_specs=[
            pl.BlockSpec(memory_space=pltpu.VMEM),
            pl.BlockSpec(memory_space=pltpu.VMEM),
            pl.BlockSpec(memory_space=pltpu.VMEM),
        ],
        out_specs=pl.BlockSpec(memory_space=pltpu.VMEM),
        out_shape=jax.ShapeDtypeStruct((B, 304, 8), jnp.float32),
    )(workp, boxes_t, labp)

    sel = misc[:, :_MAXDET, :]
    out_boxes = sel[:, :, 0:4]
    out_scores = sel[:, :, 4]
    out_labels = sel[:, :, 5].astype(jnp.int32)
    return out_boxes, out_scores, out_labels


# speedup vs baseline: 2.3405x; 1.1844x over previous
import jax
import jax.numpy as jnp
import numpy as np
from jax import lax
from jax.experimental import pallas as pl
from jax.experimental.pallas import tpu as pltpu

_SCORE_T = 0.05
_NMS_T = 0.6
_MAXDET = 300

_NB = 10
_NPAD = 20480
_ROWS = _NPAD // 128
_KSLOT = 384


def _score_body(cls_ref, cent_ref, work_ref, lab_ref):
    cls = cls_ref[:, 0]
    m = jnp.max(cls, axis=-1)
    iota_c = lax.broadcasted_iota(jnp.int32, cls.shape, 2).astype(jnp.float32)
    lab = jnp.min(jnp.where(cls == m[:, :, None], iota_c, 1e9), axis=-1)
    c0 = cent_ref[:, 0, 0]
    w = jnp.where(m > _SCORE_T, jnp.sqrt(c0 * m), -jnp.inf)
    work_ref[:, 0, 0] = w
    lab_ref[:, 0, 0] = lab


def _nms_body(w_ref, b_ref, l_ref, out_ref, ws):
    B = w_ref.shape[0]
    NEG = -jnp.inf
    out_ref[...] = jnp.full(out_ref.shape, -1.0, jnp.float32)
    ws[...] = w_ref[...]

    iota_r = lax.broadcasted_iota(jnp.int32, (1, _ROWS), 1)
    iota_l = lax.broadcasted_iota(jnp.int32, (1, 128), 1)
    iota_s = lax.broadcasted_iota(jnp.int32, (3, 128), 0) * 128 + \
        lax.broadcasted_iota(jnp.int32, (3, 128), 1)
    iota_8 = lax.broadcasted_iota(jnp.int32, (1, 8), 1)

    rm0 = [jnp.max(w_ref[b], axis=1).reshape(1, _ROWS) for b in range(B)]
    k0 = [jnp.int32(0) for _ in range(B)]
    kept0 = [[jnp.zeros((3, 128), jnp.float32) for _ in range(5)]
             for _ in range(B)]

    def flat(c):
        return tuple(c[0]) + tuple(c[1]) + tuple(x for kb in c[2] for x in kb)

    def alive_fn(k_b, rm_b):
        return (k_b < _MAXDET) & (jnp.max(rm_b) > NEG)

    def cond(state):
        ks, rms, _ = state
        a = alive_fn(ks[0], rms[0])
        for b in range(1, B):
            a = a | alive_fn(ks[b], rms[b])
        return a

    def body(state):
        ks, rms, kepts = state
        nks, nrms, nkepts = [], [], []
        for b in range(B):
            k_b, rm_b, (kx1, ky1, kx2, ky2, kar) = ks[b], rms[b], kepts[b]
            gm = jnp.max(rm_b)
            alive = (k_b < _MAXDET) & (gm > NEG)
            r = jnp.min(jnp.where(rm_b == gm, iota_r, _ROWS)).astype(jnp.int32)
            r = jnp.where(alive, r, 0)
            wrow = ws[b, pl.ds(r, 1), :]
            c = jnp.min(jnp.where(wrow == gm, iota_l, 128)).astype(jnp.int32)
            c = jnp.where(alive, c, 0)
            oh_l = iota_l == c

            def ext(ref_row):
                return jnp.sum(jnp.where(oh_l, ref_row, 0.0))

            bx1 = ext(b_ref[b, 0, pl.ds(r, 1), :])
            by1 = ext(b_ref[b, 1, pl.ds(r, 1), :])
            bx2 = ext(b_ref[b, 2, pl.ds(r, 1), :])
            by2 = ext(b_ref[b, 3, pl.ds(r, 1), :])
            blab = ext(l_ref[b, pl.ds(r, 1), :])
            bar = jnp.maximum(bx2 - bx1, 0.0) * jnp.maximum(by2 - by1, 0.0)

            ix1 = jnp.maximum(kx1, bx1)
            iy1 = jnp.maximum(ky1, by1)
            ix2 = jnp.minimum(kx2, bx2)
            iy2 = jnp.minimum(ky2, by2)
            inter = jnp.maximum(ix2 - ix1, 0.0) * jnp.maximum(iy2 - iy1, 0.0)
            union = kar + bar - inter
            iou = jnp.where(union > 0.0, inter / union, 0.0)
            hit = (iou > _NMS_T) & (iota_s < k_b)
            suppressed = jnp.max(jnp.where(hit, 1.0, 0.0)) > 0.0
            keep_it = alive & jnp.logical_not(suppressed)

            row = jnp.where(iota_8 == 0, bx1,
                  jnp.where(iota_8 == 1, by1,
                  jnp.where(iota_8 == 2, bx2,
                  jnp.where(iota_8 == 3, by2,
                  jnp.where(iota_8 == 4, gm,
                  jnp.where(iota_8 == 5, blab, 0.0))))))

            @pl.when(keep_it)
            def _():
                out_ref[b, pl.ds(k_b, 1), :] = row

            oh_s = (iota_s == k_b) & keep_it
            kx1 = jnp.where(oh_s, bx1, kx1)
            ky1 = jnp.where(oh_s, by1, ky1)
            kx2 = jnp.where(oh_s, bx2, kx2)
            ky2 = jnp.where(oh_s, by2, ky2)
            kar = jnp.where(oh_s, bar, kar)

            wrow_new = jnp.where(oh_l, NEG, wrow)

            @pl.when(alive)
            def _():
                ws[b, pl.ds(r, 1), :] = wrow_new

            nr = jnp.max(wrow_new)
            rm_b = jnp.where((iota_r == r) & alive, nr, rm_b)

            nks.append(k_b + keep_it.astype(jnp.int32))
            nrms.append(rm_b)
            nkepts.append([kx1, ky1, kx2, ky2, kar])
        return nks, nrms, nkepts

    lax.while_loop(cond, body, (k0, rm0, kept0))


@jax.jit
def kernel(boxes, classification, centerness):
    B, N, C = classification.shape
    Bn = N // _NB
    cls_r = classification.reshape(B, _NB, Bn, C)
    cent_r = centerness.reshape(B, _NB, 1, Bn)

    work, labf = pl.pallas_call(
        _score_body,
        grid=(_NB,),
        in_specs=[
            pl.BlockSpec((B, 1, Bn, C), lambda i: (0, i, 0, 0)),
            pl.BlockSpec((B, 1, 1, Bn), lambda i: (0, i, 0, 0)),
        ],
        out_specs=[
            pl.BlockSpec((B, 1, 1, Bn), lambda i: (0, i, 0, 0)),
            pl.BlockSpec((B, 1, 1, Bn), lambda i: (0, i, 0, 0)),
        ],
        out_shape=[
            jax.ShapeDtypeStruct((B, _NB, 1, Bn), jnp.float32),
            jax.ShapeDtypeStruct((B, _NB, 1, Bn), jnp.float32),
        ],
    )(cls_r, cent_r)

    work = work.reshape(B, N)
    labf = labf.reshape(B, N)
    pad = _NPAD - N
    workp = jnp.pad(work, ((0, 0), (0, pad)),
                    constant_values=-np.inf).reshape(B, _ROWS, 128)
    labp = jnp.pad(labf, ((0, 0), (0, pad))).reshape(B, _ROWS, 128)
    boxes_t = jnp.pad(boxes.transpose(0, 2, 1),
                      ((0, 0), (0, 0), (0, pad))).reshape(B, 4, _ROWS, 128)

    misc = pl.pallas_call(
        _nms_body,
        in_specs=[
            pl.BlockSpec(memory_space=pltpu.VMEM),
            pl.BlockSpec(memory_space=pltpu.VMEM),
            pl.BlockSpec(memory_space=pltpu.VMEM),
        ],
        out_specs=pl.BlockSpec(memory_space=pltpu.VMEM),
        out_shape=jax.ShapeDtypeStruct((B, 304, 8), jnp.float32),
        scratch_shapes=[pltpu.VMEM((B, _ROWS, 128), jnp.float32)],
    )(workp, boxes_t, labp)

    sel = misc[:, :_MAXDET, :]
    out_boxes = sel[:, :, 0:4]
    out_scores = sel[:, :, 4]
    out_labels = sel[:, :, 5].astype(jnp.int32)
    return out_boxes, out_scores, out_labels


# speedup vs baseline: 5.8763x; 2.5107x over previous
import jax
import jax.numpy as jnp
import numpy as np
from jax import lax
from jax.experimental import pallas as pl
from jax.experimental.pallas import tpu as pltpu

_SCORE_T = 0.05
_NMS_T = 0.6
_MAXDET = 300

_NB = 10
_NPAD = 20480
_ROWS = _NPAD // 128
_KSLOT = 384


def _score_body(cls_ref, cent_ref, work_ref, lab_ref):
    cls = cls_ref[:, 0]
    m = jnp.max(cls, axis=-1)
    iota_c = lax.broadcasted_iota(jnp.int32, cls.shape, 2).astype(jnp.float32)
    lab = jnp.min(jnp.where(cls == m[:, :, None], iota_c, 1e9), axis=-1)
    c0 = cent_ref[:, 0, 0]
    w = jnp.where(m > _SCORE_T, jnp.sqrt(c0 * m), -jnp.inf)
    work_ref[:, 0, 0] = w
    lab_ref[:, 0, 0] = lab


def _nms_body(w_ref, b_ref, l_ref, out_ref, ws):
    B = w_ref.shape[0]
    NEG = -jnp.inf
    out_ref[...] = jnp.full(out_ref.shape, -1.0, jnp.float32)
    ws[...] = w_ref[...]

    iota_r = lax.broadcasted_iota(jnp.int32, (B, _ROWS), 1).astype(jnp.float32)
    iota_l = lax.broadcasted_iota(jnp.int32, (B, 128), 1).astype(jnp.float32)
    iota_s = lax.broadcasted_iota(
        jnp.int32, (B, _KSLOT), 1).astype(jnp.float32)
    iota_8 = lax.broadcasted_iota(jnp.int32, (B, 8), 1)

    def rmax(v):
        return jnp.max(v, axis=1, keepdims=True)

    def rmin(v):
        return jnp.min(v, axis=1, keepdims=True)

    def rsum(v):
        return jnp.sum(v, axis=1, keepdims=True)

    rm0 = jnp.max(w_ref[...], axis=2)
    gm0 = rmax(rm0)
    kept0 = [jnp.zeros((B, _KSLOT), jnp.float32) for _ in range(5)]
    k0 = jnp.zeros((B, 1), jnp.float32)
    state0 = (jnp.int32(0), jnp.int32(0), k0, gm0, rm0) + tuple(kept0)

    def cond(state):
        _, _, k, gm, *_ = state
        alive = (k < float(_MAXDET)) & (gm > NEG)
        return jnp.max(jnp.where(alive, 1.0, 0.0)) > 0.0

    def body(state):
        ks0, ks1, k, gm, rm, kx1, ky1, kx2, ky2, kar = state
        alive = (k < float(_MAXDET)) & (gm > NEG)
        r = rmin(jnp.where(rm == gm, iota_r, 3e5))
        r_s0 = jnp.max(r[0:1, :]).astype(jnp.int32)
        r_s1 = jnp.max(r[1:2, :]).astype(jnp.int32)

        def rows(f):
            return jnp.concatenate([f(0, r_s0), f(1, r_s1)], axis=0)

        wrow = rows(lambda b, r_s: ws[b, pl.ds(r_s, 1), :])
        x1r = rows(lambda b, r_s: b_ref[b, 0, pl.ds(r_s, 1), :])
        y1r = rows(lambda b, r_s: b_ref[b, 1, pl.ds(r_s, 1), :])
        x2r = rows(lambda b, r_s: b_ref[b, 2, pl.ds(r_s, 1), :])
        y2r = rows(lambda b, r_s: b_ref[b, 3, pl.ds(r_s, 1), :])
        labr = rows(lambda b, r_s: l_ref[b, pl.ds(r_s, 1), :])

        c = rmin(jnp.where(wrow == gm, iota_l, 3e5))
        oh_l = iota_l == c

        def ext(v):
            return rsum(jnp.where(oh_l, v, 0.0))

        bx1, by1, bx2, by2, blab = (ext(x1r), ext(y1r), ext(x2r), ext(y2r),
                                    ext(labr))
        bar = jnp.maximum(bx2 - bx1, 0.0) * jnp.maximum(by2 - by1, 0.0)

        ix1 = jnp.maximum(kx1, bx1)
        iy1 = jnp.maximum(ky1, by1)
        ix2 = jnp.minimum(kx2, bx2)
        iy2 = jnp.minimum(ky2, by2)
        inter = jnp.maximum(ix2 - ix1, 0.0) * jnp.maximum(iy2 - iy1, 0.0)
        union = kar + bar - inter
        iou = jnp.where(union > 0.0, inter / union, 0.0)
        hit = (iou > _NMS_T) & (iota_s < k)
        suppressed = rmax(jnp.where(hit, 1.0, 0.0)) > 0.0
        keep = alive & jnp.logical_not(suppressed)

        row = jnp.where(iota_8 == 0, bx1,
              jnp.where(iota_8 == 1, by1,
              jnp.where(iota_8 == 2, bx2,
              jnp.where(iota_8 == 3, by2,
              jnp.where(iota_8 == 4, gm,
              jnp.where(iota_8 == 5, blab, 0.0))))))
        row = jnp.where(keep, row, -1.0)
        out_ref[0, pl.ds(ks0, 1), :] = row[0:1, :]
        out_ref[1, pl.ds(ks1, 1), :] = row[1:2, :]

        oh_s = (iota_s == k) & keep
        kx1 = jnp.where(oh_s, bx1, kx1)
        ky1 = jnp.where(oh_s, by1, ky1)
        kx2 = jnp.where(oh_s, bx2, kx2)
        ky2 = jnp.where(oh_s, by2, ky2)
        kar = jnp.where(oh_s, bar, kar)

        wrow_new = jnp.where(oh_l, NEG, wrow)
        ws[0, pl.ds(r_s0, 1), :] = wrow_new[0:1, :]
        ws[1, pl.ds(r_s1, 1), :] = wrow_new[1:2, :]
        nr = rmax(wrow_new)
        rm = jnp.where(iota_r == r, nr, rm)
        gm_n = rmax(rm)

        keep_f = jnp.where(keep, 1.0, 0.0)
        ks0_n = ks0 + jnp.max(keep_f[0:1, :]).astype(jnp.int32)
        ks1_n = ks1 + jnp.max(keep_f[1:2, :]).astype(jnp.int32)
        return (ks0_n, ks1_n, k + keep_f, gm_n, rm,
                kx1, ky1, kx2, ky2, kar)

    lax.while_loop(cond, body, state0)


@jax.jit
def kernel(boxes, classification, centerness):
    B, N, C = classification.shape
    Bn = N // _NB
    cls_r = classification.reshape(B, _NB, Bn, C)
    cent_r = centerness.reshape(B, _NB, 1, Bn)

    work, labf = pl.pallas_call(
        _score_body,
        grid=(_NB,),
        in_specs=[
            pl.BlockSpec((B, 1, Bn, C), lambda i: (0, i, 0, 0)),
            pl.BlockSpec((B, 1, 1, Bn), lambda i: (0, i, 0, 0)),
        ],
        out_specs=[
            pl.BlockSpec((B, 1, 1, Bn), lambda i: (0, i, 0, 0)),
            pl.BlockSpec((B, 1, 1, Bn), lambda i: (0, i, 0, 0)),
        ],
        out_shape=[
            jax.ShapeDtypeStruct((B, _NB, 1, Bn), jnp.float32),
            jax.ShapeDtypeStruct((B, _NB, 1, Bn), jnp.float32),
        ],
    )(cls_r, cent_r)

    work = work.reshape(B, N)
    labf = labf.reshape(B, N)
    pad = _NPAD - N
    workp = jnp.pad(work, ((0, 0), (0, pad)),
                    constant_values=-np.inf).reshape(B, _ROWS, 128)
    labp = jnp.pad(labf, ((0, 0), (0, pad))).reshape(B, _ROWS, 128)
    boxes_t = jnp.pad(boxes.transpose(0, 2, 1),
                      ((0, 0), (0, 0), (0, pad))).reshape(B, 4, _ROWS, 128)

    misc = pl.pallas_call(
        _nms_body,
        in_specs=[
            pl.BlockSpec(memory_space=pltpu.VMEM),
            pl.BlockSpec(memory_space=pltpu.VMEM),
            pl.BlockSpec(memory_space=pltpu.VMEM),
        ],
        out_specs=pl.BlockSpec(memory_space=pltpu.VMEM),
        out_shape=jax.ShapeDtypeStruct((B, 304, 8), jnp.float32),
        scratch_shapes=[pltpu.VMEM((B, _ROWS, 128), jnp.float32)],
    )(workp, boxes_t, labp)

    sel = misc[:, :_MAXDET, :]
    out_boxes = sel[:, :, 0:4]
    out_scores = sel[:, :, 4]
    out_labels = sel[:, :, 5].astype(jnp.int32)
    return out_boxes, out_scores, out_labels
